# Initial kernel scaffold; baseline (speedup 1.0000x reference)
import jax
import jax.numpy as jnp
from jax.experimental import pallas as pl


def kernel(input_labels, pos_labels, neg_labels, in_embed_weight):
    raise NotImplementedError("write your pallas kernel here")


# speedup vs baseline: 6.5131x; 6.5131x over previous
import functools

import jax
import jax.numpy as jnp
from jax import lax
from jax.experimental import pallas as pl
from jax.experimental.pallas import tpu as pltpu
from jax.experimental.pallas import tpu_sc as plsc

VOCAB = 1000000
D = 64
B = 16384
NPOS = 4
NNEG = 20
NCTX = NPOS + NNEG

NC = 2
NS = 16
L = 16
NW = NC * NS
BPW = B // NW
CHUNK = 16
NCHUNKS = BPW // CHUNK


def _sc_scores(input_labels, pos_flat, neg_flat, table):
    mesh = plsc.VectorSubcoreMesh(core_axis_name="c", subcore_axis_name="s")

    @functools.partial(
        pl.kernel,
        out_type=jax.ShapeDtypeStruct((NCTX, B), jnp.float32),
        mesh=mesh,
        scratch_types=[
            pltpu.VMEM((BPW,), jnp.int32),
            pltpu.VMEM((BPW * NPOS,), jnp.int32),
            pltpu.VMEM((BPW * NNEG,), jnp.int32),
            pltpu.VMEM((2, CHUNK, D), jnp.float32),
            pltpu.VMEM((2, CHUNK * NPOS, D), jnp.float32),
            pltpu.VMEM((2, CHUNK * NNEG, D), jnp.float32),
            pltpu.VMEM((NCTX, BPW), jnp.float32),
            pltpu.SemaphoreType.DMA,
            pltpu.SemaphoreType.DMA,
        ],
        compiler_params=pltpu.CompilerParams(
            use_tc_tiling_on_sc=False, needs_layout_passes=False),
    )
    def sc(in_hbm, pos_hbm, neg_hbm, table_hbm, out_hbm,
           iidx, pidx, nidx, irows, prows, nrows, sloc, sem0, sem1):
        wid = lax.axis_index("s") * NC + lax.axis_index("c")
        base = wid * BPW

        pltpu.sync_copy(in_hbm.at[pl.ds(base, BPW)], iidx)
        pltpu.sync_copy(pos_hbm.at[pl.ds(base * NPOS, BPW * NPOS)], pidx)
        pltpu.sync_copy(neg_hbm.at[pl.ds(base * NNEG, BPW * NNEG)], nidx)

        sems = (sem0, sem1)

        def copies(slot, g):
            sem = sems[slot]
            return (
                pltpu.make_async_copy(
                    table_hbm.at[iidx.at[pl.ds(g * CHUNK, CHUNK)]],
                    irows.at[slot], sem),
                pltpu.make_async_copy(
                    table_hbm.at[pidx.at[pl.ds(g * CHUNK * NPOS,
                                               CHUNK * NPOS)]],
                    prows.at[slot], sem),
                pltpu.make_async_copy(
                    table_hbm.at[nidx.at[pl.ds(g * CHUNK * NNEG,
                                               CHUNK * NNEG)]],
                    nrows.at[slot], sem),
            )

        def gather_start(slot, g):
            for c in copies(slot, g):
                c.start()

        def gather_wait(slot, g):
            for c in copies(slot, g):
                c.wait()

        lanes = lax.iota(jnp.int32, L)
        rin = lanes
        GRP = 8

        def compute(slot, g):
            ir = irows.at[slot]
            pr = prows.at[slot]
            nr = nrows.at[slot]
            specs = ([(pr, lanes * NPOS + p, 1.0) for p in range(NPOS)]
                     + [(nr, lanes * NNEG + n, -1.0) for n in range(NNEG)])
            off = g * CHUNK

            for g0 in range(0, NCTX, GRP):
                grp = specs[g0:g0 + GRP]

                def d_body(d, accs, grp=grp):
                    dcol = jnp.full((L,), d, dtype=jnp.int32)
                    ind = plsc.load_gather(ir, [rin, dcol])
                    return tuple(
                        acc + ind * plsc.load_gather(ref, [row, dcol])
                        for acc, (ref, row, _) in zip(accs, grp))

                accs = tuple(jnp.zeros((L,), jnp.float32) for _ in grp)
                accs = lax.fori_loop(0, D, d_body, accs)
                for c, (acc, (_, _, sign)) in enumerate(zip(accs, grp),
                                                        start=g0):
                    sloc[c, pl.ds(off, L)] = acc if sign > 0 else -acc

        gather_start(0, 0)

        def pair_body(i, carry):
            g0 = 2 * i
            gather_start(1, g0 + 1)
            gather_wait(0, g0)
            compute(0, g0)

            @pl.when(g0 + 2 < NCHUNKS)
            def _():
                gather_start(0, g0 + 2)

            gather_wait(1, g0 + 1)
            compute(1, g0 + 1)
            return carry

        lax.fori_loop(0, NCHUNKS // 2, pair_body, 0)
        pltpu.sync_copy(sloc, out_hbm.at[:, pl.ds(base, BPW)])

    return sc(input_labels, pos_flat, neg_flat, table)


def _tc_loss(scores):
    BLK = 2048

    def body(s_ref, o_ref):
        x = s_ref[...]
        ls = jnp.minimum(x, 0.0) - jnp.log1p(jnp.exp(-jnp.abs(x)))
        o_ref[...] = -jnp.sum(ls, axis=0)

    return pl.pallas_call(
        body,
        grid=(B // BLK,),
        in_specs=[pl.BlockSpec((NCTX, BLK), lambda i: (0, i))],
        out_specs=pl.BlockSpec((BLK,), lambda i: (i,)),
        out_shape=jax.ShapeDtypeStruct((B,), jnp.float32),
    )(scores)


def kernel(input_labels, pos_labels, neg_labels, in_embed_weight):
    input_labels = input_labels.astype(jnp.int32)
    pos_flat = pos_labels.astype(jnp.int32).reshape(-1)
    neg_flat = neg_labels.astype(jnp.int32).reshape(-1)
    scores = _sc_scores(input_labels, pos_flat, neg_flat, in_embed_weight)
    return _tc_loss(scores)
